# baseline (speedup 1.0000x reference)
import functools

import jax
import jax.numpy as jnp
from jax import lax
from jax.experimental import pallas as pl
from jax.experimental.pallas import tpu as pltpu
from jax.experimental.pallas import tpu_sc as plsc

_L = 16

_GDN = lax.GatherDimensionNumbers(
    offset_dims=(), collapsed_slice_dims=(0,), start_index_map=(0,))


def _lane_sum(x):
    lanes = lax.iota(jnp.int32, _L)
    for s in (8, 4, 2, 1):
        idx = (lanes ^ s).reshape(_L, 1)
        x = x + lax.gather(x, idx, _GDN, (1,),
                           mode=lax.GatherScatterMode.PROMISE_IN_BOUNDS)
    return x


def _build_sc_call(B, FD, P2, D):
    NC, NS = 2, 16
    NW = NC * NS
    rows_per_worker = B // NW
    R = 32
    n_chunks = rows_per_worker // R
    KD = D // _L
    NG = P2 // _L

    mesh = plsc.VectorSubcoreMesh(core_axis_name="c", subcore_axis_name="s")

    @functools.partial(
        pl.kernel,
        mesh=mesh,
        out_type=jax.ShapeDtypeStruct((B,), jnp.float32),
        scratch_types=[
            pltpu.VMEM((R, FD), jnp.float32),
            pltpu.VMEM((R, P2), jnp.float32),
            pltpu.VMEM((rows_per_worker,), jnp.float32),
            pltpu.VMEM((P2,), jnp.int32),
            pltpu.VMEM((P2,), jnp.int32),
            pltpu.VMEM((D,), jnp.float32),
        ],
    )
    def sc_kernel(emb_hbm, off1_hbm, off2_hbm, sc_hbm, w_hbm, out_hbm,
                  emb_v, sc_v, out_v, o1_v, o2_v, w_v):
        wid = lax.axis_index("s") * NC + lax.axis_index("c")
        base = wid * rows_per_worker
        pltpu.sync_copy(off1_hbm, o1_v)
        pltpu.sync_copy(off2_hbm, o2_v)
        pltpu.sync_copy(w_hbm, w_v)
        w_regs = [w_v[pl.ds(k * _L, _L)] for k in range(KD)]
        zero = jnp.zeros((_L,), jnp.float32)
        lanes = lax.iota(jnp.int32, _L)

        def chunk_body(c, _):
            row0 = base + c * R
            pltpu.sync_copy(emb_hbm.at[pl.ds(row0, R)], emb_v)
            pltpu.sync_copy(sc_hbm.at[pl.ds(row0, R)], sc_v)

            def rgrp_body(rg, _):
                def row_body(l, vec):
                    bi = rg * _L + l

                    def grp_body(g, acc):
                        gbase = pl.multiple_of(g * _L, _L)
                        o1g = o1_v[pl.ds(gbase, _L)]
                        o2g = o2_v[pl.ds(gbase, _L)]
                        sg = sc_v[bi, pl.ds(gbase, _L)]
                        acc = list(acc)
                        for j in range(_L):
                            o1 = o1g[j]
                            o2 = o2g[j]
                            s = sg[j]
                            for k in range(KD):
                                e1 = emb_v[bi, pl.ds(pl.multiple_of(o1 + k * _L, _L), _L)]
                                e2 = emb_v[bi, pl.ds(pl.multiple_of(o2 + k * _L, _L), _L)]
                                acc[k] = acc[k] + s * (e1 * e2)
                        return tuple(acc)

                    acc = lax.fori_loop(0, NG, grp_body, (zero,) * KD)
                    proj = acc[0] * w_regs[0]
                    for k in range(1, KD):
                        proj = proj + acc[k] * w_regs[k]
                    tot = _lane_sum(proj)
                    return jnp.where(lanes == l, tot, vec)

                vec = lax.fori_loop(0, _L, row_body, zero)
                out_v[pl.ds(pl.multiple_of(c * R + rg * _L, _L), _L)] = vec
                return 0

            lax.fori_loop(0, R // _L, rgrp_body, 0)
            return 0

        lax.fori_loop(0, n_chunks, chunk_body, 0)
        pltpu.sync_copy(out_v, out_hbm.at[pl.ds(base, rows_per_worker)])

    return sc_kernel


def kernel(embeddings, selected_pairs, interaction_scores, W, b):
    B, F, D = embeddings.shape
    P = selected_pairs.shape[0]
    P2 = ((P + _L - 1) // _L) * _L
    emb2 = embeddings.reshape(B, F * D)
    off1 = jnp.zeros((P2,), jnp.int32).at[:P].set(
        selected_pairs[:, 0].astype(jnp.int32) * D)
    off2 = jnp.zeros((P2,), jnp.int32).at[:P].set(
        selected_pairs[:, 1].astype(jnp.int32) * D)
    scores = jnp.pad(interaction_scores, ((0, 0), (0, P2 - P)))
    wv = W.reshape(D)
    sc_call = _build_sc_call(B, F * D, P2, D)
    out = sc_call(emb2, off1, off2, scores, wv)
    return out.reshape(B, 1) + b
